# Initial kernel scaffold; baseline (speedup 1.0000x reference)
import functools

import jax
import jax.numpy as jnp
from jax import lax
from jax.experimental import pallas as pl
from jax.experimental.pallas import tpu as pltpu
from jax.experimental.pallas import tpu_sc as plsc

N = 10000
E = 320000
DIM = 256
NC = 10
NG = 64

NTILE = 16
K = 128
CH = 160
EPAD = NTILE * CH * K
NPAD = 10240
ROWS_PER_TILE = NPAD // NTILE
OUT_ROWS_PER_TILE = N // NTILE


def _make_sc_segment_sum(fh):
    mesh = plsc.VectorSubcoreMesh(core_axis_name="c", subcore_axis_name="s")

    @functools.partial(
        pl.kernel,
        out_type=(
            jax.ShapeDtypeStruct((N, fh), jnp.float32),
            jax.ShapeDtypeStruct((N, fh), jnp.float32),
        ),
        mesh=mesh,
        scratch_types=[
            pltpu.VMEM((CH, K), jnp.int32),
            pltpu.VMEM((CH, K), jnp.int32),
            pltpu.VMEM((K, fh), jnp.float32),
            pltpu.VMEM((K, fh), jnp.float32),
            pltpu.VMEM_SHARED((NPAD, fh), jnp.float32),
            pltpu.SemaphoreType.DMA,
            pltpu.SemaphoreType.DMA,
        ],
    )
    def sc_seg(xl_hbm, xh_hbm, src_hbm, dst_hbm, zeros_hbm,
               outl_hbm, outh_hbm,
               srcv, dstv, bufa, bufb, agg, sema, semb):
        cid = lax.axis_index("c")
        sid = lax.axis_index("s")

        pltpu.sync_copy(zeros_hbm, agg.at[pl.ds(sid * ROWS_PER_TILE, ROWS_PER_TILE)])
        pltpu.sync_copy(src_hbm.at[sid], srcv)
        pltpu.sync_copy(dst_hbm.at[sid], dstv)
        plsc.subcore_barrier()

        def run(x_hbm, out_hbm):
            pltpu.async_copy(x_hbm.at[srcv.at[0]], bufa, sema)

            @pl.loop(0, CH // 2)
            def _(jj):
                j = jj * 2
                pltpu.make_async_copy(x_hbm.at[srcv.at[j]], bufa, sema).wait()
                pltpu.async_copy(x_hbm.at[srcv.at[j + 1]], bufb, semb)
                pltpu.sync_copy(bufa, agg.at[dstv.at[j]], add=True)
                pltpu.make_async_copy(x_hbm.at[srcv.at[j + 1]], bufb, semb).wait()

                @pl.when(jj + 1 < CH // 2)
                def _():
                    pltpu.async_copy(x_hbm.at[srcv.at[j + 2]], bufa, sema)

                pltpu.sync_copy(bufb, agg.at[dstv.at[j + 1]], add=True)

            plsc.subcore_barrier()
            base = sid * OUT_ROWS_PER_TILE
            pltpu.sync_copy(agg.at[pl.ds(base, OUT_ROWS_PER_TILE)],
                            out_hbm.at[pl.ds(base, OUT_ROWS_PER_TILE)])

        @pl.when(cid == 0)
        def _():
            run(xl_hbm, outl_hbm)

        @pl.when(cid == 1)
        def _():
            run(xh_hbm, outh_hbm)

    return sc_seg


_sc_seg_64 = _make_sc_segment_sum(64)
_sc_seg_128 = _make_sc_segment_sum(128)


def _tc_layer_body(residual, aggl_ref, aggh_ref, xl_ref, xh_ref,
                   wrel_ref, brel_ref, wroot_ref, g_ref, bt_ref,
                   outl_ref, outh_ref):
    fh = aggl_ref.shape[1]
    f32 = jnp.float32
    h = (jnp.dot(aggl_ref[...], wrel_ref[:fh, :], preferred_element_type=f32)
         + jnp.dot(aggh_ref[...], wrel_ref[fh:, :], preferred_element_type=f32)
         + jnp.dot(xl_ref[...], wroot_ref[:fh, :], preferred_element_type=f32)
         + jnp.dot(xh_ref[...], wroot_ref[fh:, :], preferred_element_type=f32)
         + brel_ref[...])
    m = jnp.mean(h, axis=0, keepdims=True)
    c = h - m
    v = jnp.mean(c * c, axis=0, keepdims=True)
    y = jnp.maximum(c * lax.rsqrt(v + 1e-5) * g_ref[...] + bt_ref[...], 0.0)
    half = DIM // 2
    if residual:
        outl_ref[...] = y[:, :half] + xl_ref[...]
        outh_ref[...] = y[:, half:] + xh_ref[...]
    else:
        outl_ref[...] = y[:, :half]
        outh_ref[...] = y[:, half:]


def _tc_layer(aggl, aggh, xl, xh, w_rel, b_rel, w_root, gamma, beta, residual):
    out_shape = (jax.ShapeDtypeStruct((N, DIM // 2), jnp.float32),
                 jax.ShapeDtypeStruct((N, DIM // 2), jnp.float32))
    return pl.pallas_call(
        functools.partial(_tc_layer_body, residual),
        out_shape=out_shape,
    )(aggl, aggh, xl, xh, w_rel, b_rel.reshape(1, DIM), w_root,
      gamma.reshape(1, DIM), beta.reshape(1, DIM))


def _tc_pool_body(xl_ref, xh_ref, batch_ref, wl1_ref, bl1_ref,
                  wl2_ref, bl2_ref, out_ref):
    f32 = jnp.float32
    groups = lax.broadcasted_iota(jnp.int32, (NG, N), 0)
    onehot = (batch_ref[...] == groups).astype(f32)
    pooled_l = jnp.dot(onehot, xl_ref[...], preferred_element_type=f32)
    pooled_h = jnp.dot(onehot, xh_ref[...], preferred_element_type=f32)
    pooled = jnp.concatenate([pooled_l, pooled_h], axis=1)
    h = jnp.maximum(
        jnp.dot(pooled, wl1_ref[...], preferred_element_type=f32) + bl1_ref[...],
        0.0)
    o = jnp.dot(h, wl2_ref[...], preferred_element_type=f32) + bl2_ref[...]
    shifted = o - jnp.max(o, axis=-1, keepdims=True)
    lse = jnp.log(jnp.sum(jnp.exp(shifted), axis=-1, keepdims=True))
    out_ref[...] = shifted - lse


def _tc_pool(xl, xh, batch, w_l1, b_l1, w_l2, b_l2):
    return pl.pallas_call(
        _tc_pool_body,
        out_shape=jax.ShapeDtypeStruct((NG, NC), jnp.float32),
    )(xl, xh, batch.reshape(1, N).astype(jnp.int32), w_l1,
      b_l1.reshape(1, DIM), w_l2, b_l2.reshape(1, NC))


def kernel(x, edge_index, batch,
           W_rel1, b_rel1, W_root1, gamma1, beta1,
           W_rel2, b_rel2, W_root2, gamma2, beta2,
           W_rel3, b_rel3, W_root3, gamma3, beta3,
           W_rel4, b_rel4, W_root4, gamma4, beta4,
           W_l1, b_l1, W_l2, b_l2):
    src = edge_index[0].astype(jnp.int32)
    dst = edge_index[1].astype(jnp.int32)
    pad = EPAD - E
    src_tiles = jnp.concatenate(
        [src, jnp.zeros((pad,), jnp.int32)]).reshape(NTILE, CH, K)
    dst_tiles = jnp.concatenate(
        [dst, jnp.full((pad,), N, jnp.int32)]).reshape(NTILE, CH, K)

    fin_h = x.shape[1] // 2
    zeros_small = jnp.zeros((ROWS_PER_TILE, fin_h), jnp.float32)
    zeros_big = jnp.zeros((ROWS_PER_TILE, DIM // 2), jnp.float32)

    xl = x[:, :fin_h]
    xh = x[:, fin_h:]

    layers = (
        (W_rel1, b_rel1, W_root1, gamma1, beta1, False),
        (W_rel2, b_rel2, W_root2, gamma2, beta2, True),
        (W_rel3, b_rel3, W_root3, gamma3, beta3, True),
        (W_rel4, b_rel4, W_root4, gamma4, beta4, True),
    )
    for i, (w_rel, b_rel, w_root, gamma, beta, residual) in enumerate(layers):
        if i == 0:
            sc = _sc_seg_64 if fin_h == 64 else _sc_seg_128
            zeros = zeros_small
        else:
            sc = _sc_seg_128
            zeros = zeros_big
        aggl, aggh = sc(xl, xh, src_tiles, dst_tiles, zeros)
        xl, xh = _tc_layer(aggl, aggh, xl, xh, w_rel, b_rel, w_root,
                           gamma, beta, residual)

    return _tc_pool(xl, xh, batch, W_l1, b_l1, W_l2, b_l2)


# speedup vs baseline: 2.7765x; 2.7765x over previous
import functools

import jax
import jax.numpy as jnp
from jax import lax
from jax.experimental import pallas as pl
from jax.experimental.pallas import tpu as pltpu
from jax.experimental.pallas import tpu_sc as plsc

N = 10000
E = 320000
DIM = 256
NC = 10
NG = 64

NTILE = 16
K = 128
CH = 160
SB = 16
EPAD = NTILE * CH * K
NPAD = 10240
ROWS_PER_TILE = NPAD // NTILE
OUT_ROWS_PER_TILE = N // NTILE


def _sc_prologue(zeros_hbm, agg, sid):
    pltpu.sync_copy(zeros_hbm, agg.at[pl.ds(sid * ROWS_PER_TILE, ROWS_PER_TILE)])
    plsc.subcore_barrier()


def _sc_run(x_hbm, out_hbm, src_hbm, dst_hbm, srcv, dstv, bufa, bufb, agg,
            sema, semb, sid, chunk0, nchunks):
    @pl.loop(0, nchunks // SB)
    def _(sb):
        row0 = chunk0 + sb * SB
        pltpu.sync_copy(src_hbm.at[pl.ds(row0, SB)], srcv)
        pltpu.sync_copy(dst_hbm.at[pl.ds(row0, SB)], dstv)

        @pl.loop(0, SB // 2)
        def _(jj):
            j = jj * 2
            pltpu.sync_copy(x_hbm.at[srcv.at[j]], bufa)
            pltpu.sync_copy(bufa, agg.at[dstv.at[j]], add=True)
            pltpu.sync_copy(x_hbm.at[srcv.at[j + 1]], bufb)
            pltpu.sync_copy(bufb, agg.at[dstv.at[j + 1]], add=True)

    plsc.subcore_barrier()
    base = sid * ROWS_PER_TILE
    pltpu.sync_copy(agg.at[pl.ds(base, ROWS_PER_TILE)],
                    out_hbm.at[pl.ds(base, ROWS_PER_TILE)])


def _make_sc_mesh():
    return plsc.VectorSubcoreMesh(core_axis_name="c", subcore_axis_name="s")


def _make_sc_seg_featsplit():
    fh = DIM // 2

    @functools.partial(
        pl.kernel,
        out_type=(
            jax.ShapeDtypeStruct((NPAD, fh), jnp.float32),
            jax.ShapeDtypeStruct((NPAD, fh), jnp.float32),
        ),
        mesh=_make_sc_mesh(),
        scratch_types=[
            pltpu.VMEM((SB, K), jnp.int32),
            pltpu.VMEM((SB, K), jnp.int32),
            pltpu.VMEM((K, fh), jnp.float32),
            pltpu.VMEM((K, fh), jnp.float32),
            pltpu.VMEM_SHARED((NPAD, fh), jnp.float32),
            pltpu.SemaphoreType.DMA,
            pltpu.SemaphoreType.DMA,
        ],
    )
    def sc_seg(xl_hbm, xh_hbm, src_hbm, dst_hbm, zeros_hbm,
               outl_hbm, outh_hbm,
               srcv, dstv, bufa, bufb, agg, sema, semb):
        cid = lax.axis_index("c")
        sid = lax.axis_index("s")
        _sc_prologue(zeros_hbm, agg, sid)

        @pl.when(cid == 0)
        def _():
            _sc_run(xl_hbm, outl_hbm, src_hbm, dst_hbm, srcv, dstv,
                    bufa, bufb, agg, sema, semb, sid, sid * CH, CH)

        @pl.when(cid == 1)
        def _():
            _sc_run(xh_hbm, outh_hbm, src_hbm, dst_hbm, srcv, dstv,
                    bufa, bufb, agg, sema, semb, sid, sid * CH, CH)

    return sc_seg


def _make_sc_seg_edgesplit(fw):
    @functools.partial(
        pl.kernel,
        out_type=(
            jax.ShapeDtypeStruct((NPAD, fw), jnp.float32),
            jax.ShapeDtypeStruct((NPAD, fw), jnp.float32),
        ),
        mesh=_make_sc_mesh(),
        scratch_types=[
            pltpu.VMEM((SB, K), jnp.int32),
            pltpu.VMEM((SB, K), jnp.int32),
            pltpu.VMEM((K, fw), jnp.float32),
            pltpu.VMEM((K, fw), jnp.float32),
            pltpu.VMEM_SHARED((NPAD, fw), jnp.float32),
            pltpu.SemaphoreType.DMA,
            pltpu.SemaphoreType.DMA,
        ],
    )
    def sc_seg(x_hbm, src_hbm, dst_hbm, zeros_hbm,
               out0_hbm, out1_hbm,
               srcv, dstv, bufa, bufb, agg, sema, semb):
        cid = lax.axis_index("c")
        sid = lax.axis_index("s")
        _sc_prologue(zeros_hbm, agg, sid)

        @pl.when(cid == 0)
        def _():
            _sc_run(x_hbm, out0_hbm, src_hbm, dst_hbm, srcv, dstv,
                    bufa, bufb, agg, sema, semb, sid, sid * CH, CH // 2)

        @pl.when(cid == 1)
        def _():
            _sc_run(x_hbm, out1_hbm, src_hbm, dst_hbm, srcv, dstv,
                    bufa, bufb, agg, sema, semb, sid,
                    sid * CH + CH // 2, CH // 2)

    return sc_seg


_sc_seg_feat = _make_sc_seg_featsplit()
_sc_seg_edge = _make_sc_seg_edgesplit(128)


def _tc_layer1_body(p0_ref, p1_ref, x_ref, wrel_ref, brel_ref, wroot_ref,
                    g_ref, bt_ref, outl_ref, outh_ref):
    f32 = jnp.float32
    agg = p0_ref[:N, :] + p1_ref[:N, :]
    h = (jnp.dot(agg, wrel_ref[...], preferred_element_type=f32)
         + jnp.dot(x_ref[...], wroot_ref[...], preferred_element_type=f32)
         + brel_ref[...])
    m = jnp.mean(h, axis=0, keepdims=True)
    c = h - m
    v = jnp.mean(c * c, axis=0, keepdims=True)
    y = jnp.maximum(c * lax.rsqrt(v + 1e-5) * g_ref[...] + bt_ref[...], 0.0)
    half = DIM // 2
    outl_ref[...] = y[:, :half]
    outh_ref[...] = y[:, half:]


def _tc_layer1(p0, p1, x, w_rel, b_rel, w_root, gamma, beta):
    out_shape = (jax.ShapeDtypeStruct((N, DIM // 2), jnp.float32),
                 jax.ShapeDtypeStruct((N, DIM // 2), jnp.float32))
    return pl.pallas_call(
        _tc_layer1_body,
        out_shape=out_shape,
    )(p0, p1, x, w_rel, b_rel.reshape(1, DIM), w_root,
      gamma.reshape(1, DIM), beta.reshape(1, DIM))


def _tc_layer_body(residual, aggl_ref, aggh_ref, xl_ref, xh_ref,
                   wrel_ref, brel_ref, wroot_ref, g_ref, bt_ref,
                   outl_ref, outh_ref):
    fh = aggl_ref.shape[1]
    f32 = jnp.float32
    h = (jnp.dot(aggl_ref[:N, :], wrel_ref[:fh, :], preferred_element_type=f32)
         + jnp.dot(aggh_ref[:N, :], wrel_ref[fh:, :], preferred_element_type=f32)
         + jnp.dot(xl_ref[...], wroot_ref[:fh, :], preferred_element_type=f32)
         + jnp.dot(xh_ref[...], wroot_ref[fh:, :], preferred_element_type=f32)
         + brel_ref[...])
    m = jnp.mean(h, axis=0, keepdims=True)
    c = h - m
    v = jnp.mean(c * c, axis=0, keepdims=True)
    y = jnp.maximum(c * lax.rsqrt(v + 1e-5) * g_ref[...] + bt_ref[...], 0.0)
    half = DIM // 2
    if residual:
        outl_ref[...] = y[:, :half] + xl_ref[...]
        outh_ref[...] = y[:, half:] + xh_ref[...]
    else:
        outl_ref[...] = y[:, :half]
        outh_ref[...] = y[:, half:]


def _tc_layer(aggl, aggh, xl, xh, w_rel, b_rel, w_root, gamma, beta, residual):
    out_shape = (jax.ShapeDtypeStruct((N, DIM // 2), jnp.float32),
                 jax.ShapeDtypeStruct((N, DIM // 2), jnp.float32))
    return pl.pallas_call(
        functools.partial(_tc_layer_body, residual),
        out_shape=out_shape,
    )(aggl, aggh, xl, xh, w_rel, b_rel.reshape(1, DIM), w_root,
      gamma.reshape(1, DIM), beta.reshape(1, DIM))


def _tc_pool_body(xl_ref, xh_ref, batch_ref, wl1_ref, bl1_ref,
                  wl2_ref, bl2_ref, out_ref):
    f32 = jnp.float32
    groups = lax.broadcasted_iota(jnp.int32, (NG, N), 0)
    onehot = (batch_ref[...] == groups).astype(f32)
    pooled_l = jnp.dot(onehot, xl_ref[...], preferred_element_type=f32)
    pooled_h = jnp.dot(onehot, xh_ref[...], preferred_element_type=f32)
    pooled = jnp.concatenate([pooled_l, pooled_h], axis=1)
    h = jnp.maximum(
        jnp.dot(pooled, wl1_ref[...], preferred_element_type=f32) + bl1_ref[...],
        0.0)
    o = jnp.dot(h, wl2_ref[...], preferred_element_type=f32) + bl2_ref[...]
    shifted = o - jnp.max(o, axis=-1, keepdims=True)
    lse = jnp.log(jnp.sum(jnp.exp(shifted), axis=-1, keepdims=True))
    out_ref[...] = shifted - lse


def _tc_pool(xl, xh, batch, w_l1, b_l1, w_l2, b_l2):
    return pl.pallas_call(
        _tc_pool_body,
        out_shape=jax.ShapeDtypeStruct((NG, NC), jnp.float32),
    )(xl, xh, batch.reshape(1, N).astype(jnp.int32), w_l1,
      b_l1.reshape(1, DIM), w_l2, b_l2.reshape(1, NC))


def kernel(x, edge_index, batch,
           W_rel1, b_rel1, W_root1, gamma1, beta1,
           W_rel2, b_rel2, W_root2, gamma2, beta2,
           W_rel3, b_rel3, W_root3, gamma3, beta3,
           W_rel4, b_rel4, W_root4, gamma4, beta4,
           W_l1, b_l1, W_l2, b_l2):
    src = edge_index[0].astype(jnp.int32)
    dst = edge_index[1].astype(jnp.int32)
    pad = EPAD - E
    src_tiles = jnp.concatenate(
        [src, jnp.zeros((pad,), jnp.int32)]).reshape(NTILE * CH, K)
    dst_tiles = jnp.concatenate(
        [dst, jnp.full((pad,), N, jnp.int32)]).reshape(NTILE * CH, K)

    zeros = jnp.zeros((ROWS_PER_TILE, DIM // 2), jnp.float32)

    p0, p1 = _sc_seg_edge(x, src_tiles, dst_tiles, zeros)
    xl, xh = _tc_layer1(p0, p1, x, W_rel1, b_rel1, W_root1, gamma1, beta1)

    layers = (
        (W_rel2, b_rel2, W_root2, gamma2, beta2),
        (W_rel3, b_rel3, W_root3, gamma3, beta3),
        (W_rel4, b_rel4, W_root4, gamma4, beta4),
    )
    for w_rel, b_rel, w_root, gamma, beta in layers:
        aggl, aggh = _sc_seg_feat(xl, xh, src_tiles, dst_tiles, zeros)
        xl, xh = _tc_layer(aggl, aggh, xl, xh, w_rel, b_rel, w_root,
                           gamma, beta, True)

    return _tc_pool(xl, xh, batch, W_l1, b_l1, W_l2, b_l2)


# speedup vs baseline: 3.3111x; 1.1926x over previous
import functools

import jax
import jax.numpy as jnp
from jax import lax
from jax.experimental import pallas as pl
from jax.experimental.pallas import tpu as pltpu
from jax.experimental.pallas import tpu_sc as plsc

N = 10000
E = 320000
DIM = 256
NC = 10
NG = 64

NTILE = 16
K = 64
CH = 320
SBI = 32
NBUF = 4
EPAD = NTILE * CH * K
NPAD = 10240
ROWS_PER_TILE = NPAD // NTILE
OUT_ROWS_PER_TILE = N // NTILE


def _sc_prologue(zeros_hbm, agg, sid):
    pltpu.sync_copy(zeros_hbm, agg.at[pl.ds(sid * ROWS_PER_TILE, ROWS_PER_TILE)])
    plsc.subcore_barrier()


def _sc_run(x_hbm, out_hbm, src_hbm, dst_hbm, srcv, dstv, bufs, gsems, ssems,
            agg, sid, chunk0, nchunks):
    def gather(j, b):
        return pltpu.make_async_copy(x_hbm.at[srcv.at[j]], bufs[b], gsems[b])

    def scatter(j, b):
        return pltpu.make_async_copy(bufs[b], agg.at[dstv.at[j]], ssems[b])

    @pl.loop(0, nchunks // SBI)
    def _(blk):
        row0 = chunk0 + blk * SBI
        pltpu.sync_copy(src_hbm.at[pl.ds(row0, SBI)], srcv)
        pltpu.sync_copy(dst_hbm.at[pl.ds(row0, SBI)], dstv)
        for b in range(NBUF):
            gather(b, b).start()

        @pl.loop(0, SBI // NBUF - 1)
        def _(g):
            for b in range(NBUF):
                j = g * NBUF + b
                gather(j, b).wait()
                scatter(j, b).start(add=True)
            for b in range(NBUF):
                j = g * NBUF + b
                scatter(j, b).wait()
                gather(j + NBUF, b).start()

        g_last = SBI // NBUF - 1
        for b in range(NBUF):
            j = g_last * NBUF + b
            gather(j, b).wait()
            scatter(j, b).start(add=True)
        for b in range(NBUF):
            j = g_last * NBUF + b
            scatter(j, b).wait()

    plsc.subcore_barrier()
    base = sid * ROWS_PER_TILE
    pltpu.sync_copy(agg.at[pl.ds(base, ROWS_PER_TILE)],
                    out_hbm.at[pl.ds(base, ROWS_PER_TILE)])


def _make_sc_mesh():
    return plsc.VectorSubcoreMesh(core_axis_name="c", subcore_axis_name="s")


def _make_sc_seg_featsplit():
    fh = DIM // 2

    @functools.partial(
        pl.kernel,
        out_type=(
            jax.ShapeDtypeStruct((NPAD, fh), jnp.float32),
            jax.ShapeDtypeStruct((NPAD, fh), jnp.float32),
        ),
        mesh=_make_sc_mesh(),
        scratch_types=(
            [pltpu.VMEM((SBI, K), jnp.int32),
             pltpu.VMEM((SBI, K), jnp.int32)]
            + [pltpu.VMEM((K, fh), jnp.float32)] * NBUF
            + [pltpu.SemaphoreType.DMA] * (2 * NBUF)
            + [pltpu.VMEM_SHARED((NPAD, fh), jnp.float32)]
        ),
    )
    def sc_seg(xl_hbm, xh_hbm, src_hbm, dst_hbm, zeros_hbm,
               outl_hbm, outh_hbm,
               srcv, dstv, b0, b1, b2, b3,
               gs0, gs1, gs2, gs3, ss0, ss1, ss2, ss3, agg):
        bufs = (b0, b1, b2, b3)
        gsems = (gs0, gs1, gs2, gs3)
        ssems = (ss0, ss1, ss2, ss3)
        cid = lax.axis_index("c")
        sid = lax.axis_index("s")
        _sc_prologue(zeros_hbm, agg, sid)

        @pl.when(cid == 0)
        def _():
            _sc_run(xl_hbm, outl_hbm, src_hbm, dst_hbm, srcv, dstv,
                    bufs, gsems, ssems, agg, sid, sid * CH, CH)

        @pl.when(cid == 1)
        def _():
            _sc_run(xh_hbm, outh_hbm, src_hbm, dst_hbm, srcv, dstv,
                    bufs, gsems, ssems, agg, sid, sid * CH, CH)

    return sc_seg


def _make_sc_seg_edgesplit(fw):
    @functools.partial(
        pl.kernel,
        out_type=(
            jax.ShapeDtypeStruct((NPAD, fw), jnp.float32),
            jax.ShapeDtypeStruct((NPAD, fw), jnp.float32),
        ),
        mesh=_make_sc_mesh(),
        scratch_types=(
            [pltpu.VMEM((SBI, K), jnp.int32),
             pltpu.VMEM((SBI, K), jnp.int32)]
            + [pltpu.VMEM((K, fw), jnp.float32)] * NBUF
            + [pltpu.SemaphoreType.DMA] * (2 * NBUF)
            + [pltpu.VMEM_SHARED((NPAD, fw), jnp.float32)]
        ),
    )
    def sc_seg(x_hbm, src_hbm, dst_hbm, zeros_hbm,
               out0_hbm, out1_hbm,
               srcv, dstv, b0, b1, b2, b3,
               gs0, gs1, gs2, gs3, ss0, ss1, ss2, ss3, agg):
        bufs = (b0, b1, b2, b3)
        gsems = (gs0, gs1, gs2, gs3)
        ssems = (ss0, ss1, ss2, ss3)
        cid = lax.axis_index("c")
        sid = lax.axis_index("s")
        _sc_prologue(zeros_hbm, agg, sid)

        @pl.when(cid == 0)
        def _():
            _sc_run(x_hbm, out0_hbm, src_hbm, dst_hbm, srcv, dstv,
                    bufs, gsems, ssems, agg, sid, sid * CH, CH // 2)

        @pl.when(cid == 1)
        def _():
            _sc_run(x_hbm, out1_hbm, src_hbm, dst_hbm, srcv, dstv,
                    bufs, gsems, ssems, agg, sid,
                    sid * CH + CH // 2, CH // 2)

    return sc_seg


_sc_seg_feat = _make_sc_seg_featsplit()
_sc_seg_edge = _make_sc_seg_edgesplit(128)


def _tc_layer1_body(p0_ref, p1_ref, x_ref, wrel_ref, brel_ref, wroot_ref,
                    g_ref, bt_ref, outl_ref, outh_ref):
    f32 = jnp.float32
    agg = p0_ref[:N, :] + p1_ref[:N, :]
    h = (jnp.dot(agg, wrel_ref[...], preferred_element_type=f32)
         + jnp.dot(x_ref[...], wroot_ref[...], preferred_element_type=f32)
         + brel_ref[...])
    m = jnp.mean(h, axis=0, keepdims=True)
    c = h - m
    v = jnp.mean(c * c, axis=0, keepdims=True)
    y = jnp.maximum(c * lax.rsqrt(v + 1e-5) * g_ref[...] + bt_ref[...], 0.0)
    half = DIM // 2
    outl_ref[...] = y[:, :half]
    outh_ref[...] = y[:, half:]


def _tc_layer1(p0, p1, x, w_rel, b_rel, w_root, gamma, beta):
    out_shape = (jax.ShapeDtypeStruct((N, DIM // 2), jnp.float32),
                 jax.ShapeDtypeStruct((N, DIM // 2), jnp.float32))
    return pl.pallas_call(
        _tc_layer1_body,
        out_shape=out_shape,
    )(p0, p1, x, w_rel, b_rel.reshape(1, DIM), w_root,
      gamma.reshape(1, DIM), beta.reshape(1, DIM))


def _tc_layer_body(residual, aggl_ref, aggh_ref, xl_ref, xh_ref,
                   wrel_ref, brel_ref, wroot_ref, g_ref, bt_ref,
                   outl_ref, outh_ref):
    fh = aggl_ref.shape[1]
    f32 = jnp.float32
    h = (jnp.dot(aggl_ref[:N, :], wrel_ref[:fh, :], preferred_element_type=f32)
         + jnp.dot(aggh_ref[:N, :], wrel_ref[fh:, :], preferred_element_type=f32)
         + jnp.dot(xl_ref[...], wroot_ref[:fh, :], preferred_element_type=f32)
         + jnp.dot(xh_ref[...], wroot_ref[fh:, :], preferred_element_type=f32)
         + brel_ref[...])
    m = jnp.mean(h, axis=0, keepdims=True)
    c = h - m
    v = jnp.mean(c * c, axis=0, keepdims=True)
    y = jnp.maximum(c * lax.rsqrt(v + 1e-5) * g_ref[...] + bt_ref[...], 0.0)
    half = DIM // 2
    if residual:
        outl_ref[...] = y[:, :half] + xl_ref[...]
        outh_ref[...] = y[:, half:] + xh_ref[...]
    else:
        outl_ref[...] = y[:, :half]
        outh_ref[...] = y[:, half:]


def _tc_layer(aggl, aggh, xl, xh, w_rel, b_rel, w_root, gamma, beta, residual):
    out_shape = (jax.ShapeDtypeStruct((N, DIM // 2), jnp.float32),
                 jax.ShapeDtypeStruct((N, DIM // 2), jnp.float32))
    return pl.pallas_call(
        functools.partial(_tc_layer_body, residual),
        out_shape=out_shape,
    )(aggl, aggh, xl, xh, w_rel, b_rel.reshape(1, DIM), w_root,
      gamma.reshape(1, DIM), beta.reshape(1, DIM))


def _tc_pool_body(xl_ref, xh_ref, batch_ref, wl1_ref, bl1_ref,
                  wl2_ref, bl2_ref, out_ref):
    f32 = jnp.float32
    groups = lax.broadcasted_iota(jnp.int32, (NG, N), 0)
    onehot = (batch_ref[...] == groups).astype(f32)
    pooled_l = jnp.dot(onehot, xl_ref[...], preferred_element_type=f32)
    pooled_h = jnp.dot(onehot, xh_ref[...], preferred_element_type=f32)
    pooled = jnp.concatenate([pooled_l, pooled_h], axis=1)
    h = jnp.maximum(
        jnp.dot(pooled, wl1_ref[...], preferred_element_type=f32) + bl1_ref[...],
        0.0)
    o = jnp.dot(h, wl2_ref[...], preferred_element_type=f32) + bl2_ref[...]
    shifted = o - jnp.max(o, axis=-1, keepdims=True)
    lse = jnp.log(jnp.sum(jnp.exp(shifted), axis=-1, keepdims=True))
    out_ref[...] = shifted - lse


def _tc_pool(xl, xh, batch, w_l1, b_l1, w_l2, b_l2):
    return pl.pallas_call(
        _tc_pool_body,
        out_shape=jax.ShapeDtypeStruct((NG, NC), jnp.float32),
    )(xl, xh, batch.reshape(1, N).astype(jnp.int32), w_l1,
      b_l1.reshape(1, DIM), w_l2, b_l2.reshape(1, NC))


def kernel(x, edge_index, batch,
           W_rel1, b_rel1, W_root1, gamma1, beta1,
           W_rel2, b_rel2, W_root2, gamma2, beta2,
           W_rel3, b_rel3, W_root3, gamma3, beta3,
           W_rel4, b_rel4, W_root4, gamma4, beta4,
           W_l1, b_l1, W_l2, b_l2):
    src = edge_index[0].astype(jnp.int32)
    dst = edge_index[1].astype(jnp.int32)
    pad = EPAD - E
    src_tiles = jnp.concatenate(
        [src, jnp.zeros((pad,), jnp.int32)]).reshape(NTILE * CH, K)
    dst_tiles = jnp.concatenate(
        [dst, jnp.full((pad,), N, jnp.int32)]).reshape(NTILE * CH, K)

    zeros = jnp.zeros((ROWS_PER_TILE, DIM // 2), jnp.float32)

    p0, p1 = _sc_seg_edge(x, src_tiles, dst_tiles, zeros)
    xl, xh = _tc_layer1(p0, p1, x, W_rel1, b_rel1, W_root1, gamma1, beta1)

    layers = (
        (W_rel2, b_rel2, W_root2, gamma2, beta2),
        (W_rel3, b_rel3, W_root3, gamma3, beta3),
        (W_rel4, b_rel4, W_root4, gamma4, beta4),
    )
    for w_rel, b_rel, w_root, gamma, beta in layers:
        aggl, aggh = _sc_seg_feat(xl, xh, src_tiles, dst_tiles, zeros)
        xl, xh = _tc_layer(aggl, aggh, xl, xh, w_rel, b_rel, w_root,
                           gamma, beta, True)

    return _tc_pool(xl, xh, batch, W_l1, b_l1, W_l2, b_l2)


# speedup vs baseline: 3.3419x; 1.0093x over previous
import functools

import jax
import jax.numpy as jnp
from jax import lax
from jax.experimental import pallas as pl
from jax.experimental.pallas import tpu as pltpu
from jax.experimental.pallas import tpu_sc as plsc

N = 10000
E = 320000
DIM = 256
NC = 10
NG = 64

NTILE = 16
K = 64
CH = 320
SBI = 32
NBUF = 4
EPAD = NTILE * CH * K
NPAD = 10240
ROWS_PER_TILE = NPAD // NTILE
OUT_ROWS_PER_TILE = N // NTILE


def _sc_prologue(zeros_hbm, agg, sid):
    pltpu.sync_copy(zeros_hbm, agg.at[pl.ds(sid * ROWS_PER_TILE, ROWS_PER_TILE)])
    plsc.subcore_barrier()


def _sc_run(x_hbm, out_hbm, src_hbm, dst_hbm, srcv, dstv, bufs, gsems, ssems,
            agg, sid, chunk0, nchunks):
    def gather(j, b):
        return pltpu.make_async_copy(x_hbm.at[srcv.at[j]], bufs[b], gsems[b])

    def scatter(j, b):
        return pltpu.make_async_copy(bufs[b], agg.at[dstv.at[j]], ssems[b])

    @pl.loop(0, nchunks // SBI)
    def _(blk):
        row0 = chunk0 + blk * SBI
        pltpu.sync_copy(src_hbm.at[pl.ds(row0, SBI)], srcv)
        pltpu.sync_copy(dst_hbm.at[pl.ds(row0, SBI)], dstv)
        gather(0, 0).start()
        gather(1, 1).start()
        gather(0, 0).wait()
        scatter(0, 0).start(add=True)
        gather(2, 2).start()
        gather(1, 1).wait()
        scatter(1, 1).start(add=True)
        gather(3, 3).start()

        @pl.loop(0, (SBI - 4) // NBUF)
        def _(g):
            for t in range(NBUF):
                j = g * NBUF + t + 2
                bj = (t + 2) % NBUF
                gather(j, bj).wait()
                scatter(j, bj).start(add=True)
                scatter(j - 2, t).wait()
                gather(j + 2, t).start()

        for j, bj in ((SBI - 2, (SBI - 2) % NBUF), (SBI - 1, (SBI - 1) % NBUF)):
            gather(j, bj).wait()
            scatter(j, bj).start(add=True)
        for j in range(SBI - 4, SBI):
            scatter(j, j % NBUF).wait()

    plsc.subcore_barrier()
    base = sid * ROWS_PER_TILE
    pltpu.sync_copy(agg.at[pl.ds(base, ROWS_PER_TILE)],
                    out_hbm.at[pl.ds(base, ROWS_PER_TILE)])


def _make_sc_mesh():
    return plsc.VectorSubcoreMesh(core_axis_name="c", subcore_axis_name="s")


def _make_sc_seg_featsplit():
    fh = DIM // 2

    @functools.partial(
        pl.kernel,
        out_type=(
            jax.ShapeDtypeStruct((NPAD, fh), jnp.float32),
            jax.ShapeDtypeStruct((NPAD, fh), jnp.float32),
        ),
        mesh=_make_sc_mesh(),
        scratch_types=(
            [pltpu.VMEM((SBI, K), jnp.int32),
             pltpu.VMEM((SBI, K), jnp.int32)]
            + [pltpu.VMEM((K, fh), jnp.float32)] * NBUF
            + [pltpu.SemaphoreType.DMA] * (2 * NBUF)
            + [pltpu.VMEM_SHARED((NPAD, fh), jnp.float32)]
        ),
    )
    def sc_seg(xl_hbm, xh_hbm, src_hbm, dst_hbm, zeros_hbm,
               outl_hbm, outh_hbm,
               srcv, dstv, b0, b1, b2, b3,
               gs0, gs1, gs2, gs3, ss0, ss1, ss2, ss3, agg):
        bufs = (b0, b1, b2, b3)
        gsems = (gs0, gs1, gs2, gs3)
        ssems = (ss0, ss1, ss2, ss3)
        cid = lax.axis_index("c")
        sid = lax.axis_index("s")
        _sc_prologue(zeros_hbm, agg, sid)

        @pl.when(cid == 0)
        def _():
            _sc_run(xl_hbm, outl_hbm, src_hbm, dst_hbm, srcv, dstv,
                    bufs, gsems, ssems, agg, sid, sid * CH, CH)

        @pl.when(cid == 1)
        def _():
            _sc_run(xh_hbm, outh_hbm, src_hbm, dst_hbm, srcv, dstv,
                    bufs, gsems, ssems, agg, sid, sid * CH, CH)

    return sc_seg


def _make_sc_seg_edgesplit(fw):
    @functools.partial(
        pl.kernel,
        out_type=(
            jax.ShapeDtypeStruct((NPAD, fw), jnp.float32),
            jax.ShapeDtypeStruct((NPAD, fw), jnp.float32),
        ),
        mesh=_make_sc_mesh(),
        scratch_types=(
            [pltpu.VMEM((SBI, K), jnp.int32),
             pltpu.VMEM((SBI, K), jnp.int32)]
            + [pltpu.VMEM((K, fw), jnp.float32)] * NBUF
            + [pltpu.SemaphoreType.DMA] * (2 * NBUF)
            + [pltpu.VMEM_SHARED((NPAD, fw), jnp.float32)]
        ),
    )
    def sc_seg(x_hbm, src_hbm, dst_hbm, zeros_hbm,
               out0_hbm, out1_hbm,
               srcv, dstv, b0, b1, b2, b3,
               gs0, gs1, gs2, gs3, ss0, ss1, ss2, ss3, agg):
        bufs = (b0, b1, b2, b3)
        gsems = (gs0, gs1, gs2, gs3)
        ssems = (ss0, ss1, ss2, ss3)
        cid = lax.axis_index("c")
        sid = lax.axis_index("s")
        _sc_prologue(zeros_hbm, agg, sid)

        @pl.when(cid == 0)
        def _():
            _sc_run(x_hbm, out0_hbm, src_hbm, dst_hbm, srcv, dstv,
                    bufs, gsems, ssems, agg, sid, sid * CH, CH // 2)

        @pl.when(cid == 1)
        def _():
            _sc_run(x_hbm, out1_hbm, src_hbm, dst_hbm, srcv, dstv,
                    bufs, gsems, ssems, agg, sid,
                    sid * CH + CH // 2, CH // 2)

    return sc_seg


_sc_seg_feat = _make_sc_seg_featsplit()
_sc_seg_edge = _make_sc_seg_edgesplit(128)


def _tc_layer1_body(p0_ref, p1_ref, x_ref, wrel_ref, brel_ref, wroot_ref,
                    g_ref, bt_ref, outl_ref, outh_ref):
    f32 = jnp.float32
    agg = p0_ref[:N, :] + p1_ref[:N, :]
    h = (jnp.dot(agg, wrel_ref[...], preferred_element_type=f32)
         + jnp.dot(x_ref[...], wroot_ref[...], preferred_element_type=f32)
         + brel_ref[...])
    m = jnp.mean(h, axis=0, keepdims=True)
    c = h - m
    v = jnp.mean(c * c, axis=0, keepdims=True)
    y = jnp.maximum(c * lax.rsqrt(v + 1e-5) * g_ref[...] + bt_ref[...], 0.0)
    half = DIM // 2
    outl_ref[...] = y[:, :half]
    outh_ref[...] = y[:, half:]


def _tc_layer1(p0, p1, x, w_rel, b_rel, w_root, gamma, beta):
    out_shape = (jax.ShapeDtypeStruct((N, DIM // 2), jnp.float32),
                 jax.ShapeDtypeStruct((N, DIM // 2), jnp.float32))
    return pl.pallas_call(
        _tc_layer1_body,
        out_shape=out_shape,
    )(p0, p1, x, w_rel, b_rel.reshape(1, DIM), w_root,
      gamma.reshape(1, DIM), beta.reshape(1, DIM))


def _tc_layer_body(residual, aggl_ref, aggh_ref, xl_ref, xh_ref,
                   wrel_ref, brel_ref, wroot_ref, g_ref, bt_ref,
                   outl_ref, outh_ref):
    fh = aggl_ref.shape[1]
    f32 = jnp.float32
    h = (jnp.dot(aggl_ref[:N, :], wrel_ref[:fh, :], preferred_element_type=f32)
         + jnp.dot(aggh_ref[:N, :], wrel_ref[fh:, :], preferred_element_type=f32)
         + jnp.dot(xl_ref[...], wroot_ref[:fh, :], preferred_element_type=f32)
         + jnp.dot(xh_ref[...], wroot_ref[fh:, :], preferred_element_type=f32)
         + brel_ref[...])
    m = jnp.mean(h, axis=0, keepdims=True)
    c = h - m
    v = jnp.mean(c * c, axis=0, keepdims=True)
    y = jnp.maximum(c * lax.rsqrt(v + 1e-5) * g_ref[...] + bt_ref[...], 0.0)
    half = DIM // 2
    if residual:
        outl_ref[...] = y[:, :half] + xl_ref[...]
        outh_ref[...] = y[:, half:] + xh_ref[...]
    else:
        outl_ref[...] = y[:, :half]
        outh_ref[...] = y[:, half:]


def _tc_layer(aggl, aggh, xl, xh, w_rel, b_rel, w_root, gamma, beta, residual):
    out_shape = (jax.ShapeDtypeStruct((N, DIM // 2), jnp.float32),
                 jax.ShapeDtypeStruct((N, DIM // 2), jnp.float32))
    return pl.pallas_call(
        functools.partial(_tc_layer_body, residual),
        out_shape=out_shape,
    )(aggl, aggh, xl, xh, w_rel, b_rel.reshape(1, DIM), w_root,
      gamma.reshape(1, DIM), beta.reshape(1, DIM))


def _tc_pool_body(xl_ref, xh_ref, batch_ref, wl1_ref, bl1_ref,
                  wl2_ref, bl2_ref, out_ref):
    f32 = jnp.float32
    groups = lax.broadcasted_iota(jnp.int32, (NG, N), 0)
    onehot = (batch_ref[...] == groups).astype(f32)
    pooled_l = jnp.dot(onehot, xl_ref[...], preferred_element_type=f32)
    pooled_h = jnp.dot(onehot, xh_ref[...], preferred_element_type=f32)
    pooled = jnp.concatenate([pooled_l, pooled_h], axis=1)
    h = jnp.maximum(
        jnp.dot(pooled, wl1_ref[...], preferred_element_type=f32) + bl1_ref[...],
        0.0)
    o = jnp.dot(h, wl2_ref[...], preferred_element_type=f32) + bl2_ref[...]
    shifted = o - jnp.max(o, axis=-1, keepdims=True)
    lse = jnp.log(jnp.sum(jnp.exp(shifted), axis=-1, keepdims=True))
    out_ref[...] = shifted - lse


def _tc_pool(xl, xh, batch, w_l1, b_l1, w_l2, b_l2):
    return pl.pallas_call(
        _tc_pool_body,
        out_shape=jax.ShapeDtypeStruct((NG, NC), jnp.float32),
    )(xl, xh, batch.reshape(1, N).astype(jnp.int32), w_l1,
      b_l1.reshape(1, DIM), w_l2, b_l2.reshape(1, NC))


def kernel(x, edge_index, batch,
           W_rel1, b_rel1, W_root1, gamma1, beta1,
           W_rel2, b_rel2, W_root2, gamma2, beta2,
           W_rel3, b_rel3, W_root3, gamma3, beta3,
           W_rel4, b_rel4, W_root4, gamma4, beta4,
           W_l1, b_l1, W_l2, b_l2):
    src = edge_index[0].astype(jnp.int32)
    dst = edge_index[1].astype(jnp.int32)
    pad = EPAD - E
    src_tiles = jnp.concatenate(
        [src, jnp.zeros((pad,), jnp.int32)]).reshape(NTILE * CH, K)
    dst_tiles = jnp.concatenate(
        [dst, jnp.full((pad,), N, jnp.int32)]).reshape(NTILE * CH, K)

    zeros = jnp.zeros((ROWS_PER_TILE, DIM // 2), jnp.float32)

    p0, p1 = _sc_seg_edge(x, src_tiles, dst_tiles, zeros)
    xl, xh = _tc_layer1(p0, p1, x, W_rel1, b_rel1, W_root1, gamma1, beta1)

    layers = (
        (W_rel2, b_rel2, W_root2, gamma2, beta2),
        (W_rel3, b_rel3, W_root3, gamma3, beta3),
        (W_rel4, b_rel4, W_root4, gamma4, beta4),
    )
    for w_rel, b_rel, w_root, gamma, beta in layers:
        aggl, aggh = _sc_seg_feat(xl, xh, src_tiles, dst_tiles, zeros)
        xl, xh = _tc_layer(aggl, aggh, xl, xh, w_rel, b_rel, w_root,
                           gamma, beta, True)

    return _tc_pool(xl, xh, batch, W_l1, b_l1, W_l2, b_l2)
